# Initial kernel scaffold; baseline (speedup 1.0000x reference)
import jax
import jax.numpy as jnp
from jax.experimental import pallas as pl


def kernel(x, edge_index, edge_attr, edge_embedding, node_attr, W_mlp1, b_mlp1, W_mlp2, b_mlp2, W_edge_proj, W_dst, W_nu, W_last):
    raise NotImplementedError("write your pallas kernel here")


# speedup vs baseline: 1.5410x; 1.5410x over previous
import functools
import math

import jax
import jax.numpy as jnp
import numpy as np
from jax import lax
from jax.experimental import pallas as pl
from jax.experimental.pallas import tpu as pltpu
from jax.experimental.pallas import tpu_sc as plsc

N = 10000
E = 320000
F = 128
A = 16
B = 16
H = 64
C = 16

NC = 2
NS = 16
L = 16

EP = E // (NC * NS)
K = 80
NCHUNK = EP // K
NP = 10240
ROWS_PER_TILE = NP // NS

_INV_SQRT20 = 1.0 / math.sqrt(20.0)


def _edge_coeff_body(ea_ref, ee_ref, w1_ref, b1_ref, w2_ref, b2_ref, wp_ref,
                     out_ref):
    ee = ee_ref[...]
    h = jnp.dot(ee, w1_ref[...], preferred_element_type=jnp.float32)
    h = h + b1_ref[...]
    h = h * jax.nn.sigmoid(h)
    w = jnp.dot(h, w2_ref[...], preferred_element_type=jnp.float32)
    w = w + b2_ref[...]
    ea = jnp.dot(ea_ref[...], wp_ref[...], preferred_element_type=jnp.float32)
    out_ref[...] = ea * w


def _edge_coeff(edge_attr, edge_embedding, W_mlp1, b_mlp1, W_mlp2, b_mlp2,
                W_edge_proj):
    BE = 2000
    grid = (E // BE,)
    return pl.pallas_call(
        _edge_coeff_body,
        grid=grid,
        in_specs=[
            pl.BlockSpec((BE, A), lambda i: (i, 0)),
            pl.BlockSpec((BE, B), lambda i: (i, 0)),
            pl.BlockSpec((B, H), lambda i: (0, 0)),
            pl.BlockSpec((1, H), lambda i: (0, 0)),
            pl.BlockSpec((H, F), lambda i: (0, 0)),
            pl.BlockSpec((1, F), lambda i: (0, 0)),
            pl.BlockSpec((A, F), lambda i: (0, 0)),
        ],
        out_specs=pl.BlockSpec((BE, F), lambda i: (i, 0)),
        out_shape=jax.ShapeDtypeStruct((E, F), jnp.float32),
    )(edge_attr, edge_embedding, W_mlp1, b_mlp1.reshape(1, H),
      W_mlp2, b_mlp2.reshape(1, F), W_edge_proj)


def _sc_agg_body(x_hbm, src_hbm, dst_hbm, coeff_hbm, zeros_hbm, out_hbm,
                 src_v, dst_v, rows_v, coef_v, acc_sh, sem):
    cid = lax.axis_index("c")
    sid = lax.axis_index("s")
    r0 = sid * ROWS_PER_TILE
    pltpu.sync_copy(zeros_hbm.at[pl.ds(r0, ROWS_PER_TILE)],
                    acc_sh.at[pl.ds(r0, ROWS_PER_TILE)])
    plsc.subcore_barrier()

    wid = cid * NS + sid
    base0 = wid * EP

    def chunk_body(ci, carry):
        base = base0 + ci * K
        pltpu.sync_copy(src_hbm.at[pl.ds(base, K)], src_v)
        pltpu.sync_copy(dst_hbm.at[pl.ds(base, K)], dst_v)
        pltpu.sync_copy(coeff_hbm.at[pl.ds(base, K)], coef_v)
        pltpu.async_copy(x_hbm.at[src_v], rows_v, sem).wait()

        def mul_body(e, c2):
            for j in range(F // L):
                sl = pl.ds(j * L, L)
                rows_v[e, sl] = rows_v[e, sl] * coef_v[e, sl]
            return c2

        lax.fori_loop(0, K, mul_body, 0, unroll=2)
        pltpu.sync_copy(rows_v, acc_sh.at[dst_v], add=True)
        return carry

    lax.fori_loop(0, NCHUNK, chunk_body, 0)
    plsc.subcore_barrier()
    pltpu.sync_copy(acc_sh.at[pl.ds(r0, ROWS_PER_TILE)],
                    out_hbm.at[cid, pl.ds(r0, ROWS_PER_TILE)])


_sc_agg = functools.partial(
    pl.kernel,
    out_type=jax.ShapeDtypeStruct((NC, NP, F), jnp.float32),
    mesh=plsc.VectorSubcoreMesh(core_axis_name="c", subcore_axis_name="s"),
    scratch_types=[
        pltpu.VMEM((K,), jnp.int32),
        pltpu.VMEM((K,), jnp.int32),
        pltpu.VMEM((K, F), jnp.float32),
        pltpu.VMEM((K, F), jnp.float32),
        pltpu.VMEM_SHARED((NP, F), jnp.float32),
        pltpu.SemaphoreType.DMA,
    ],
)(_sc_agg_body)


def _post_body(parts_ref, x_ref, na_ref, wd_ref, wnu_ref, wl_ref, out_ref):
    agg = (parts_ref[0] + parts_ref[1]) * _INV_SQRT20
    lin = jnp.dot(agg, wd_ref[...], preferred_element_type=jnp.float32) + agg
    g = lin * jax.nn.sigmoid(lin)
    xb = x_ref[...]
    nab = na_ref[...]
    acc = g
    for c in range(C):
        acc = acc + nab[:, c][:, None] * jnp.dot(
            xb, wnu_ref[c], preferred_element_type=jnp.float32)
    out_ref[...] = jnp.dot(acc, wl_ref[...],
                           preferred_element_type=jnp.float32) + acc


def _post(parts, x, node_attr, W_dst, W_nu_r, W_last):
    BN = 1000
    grid = (N // BN,)
    return pl.pallas_call(
        _post_body,
        grid=grid,
        in_specs=[
            pl.BlockSpec((NC, BN, F), lambda i: (0, i, 0),),
            pl.BlockSpec((BN, F), lambda i: (i, 0)),
            pl.BlockSpec((BN, C), lambda i: (i, 0)),
            pl.BlockSpec((F, F), lambda i: (0, 0)),
            pl.BlockSpec((C, F, F), lambda i: (0, 0, 0)),
            pl.BlockSpec((F, F), lambda i: (0, 0)),
        ],
        out_specs=pl.BlockSpec((BN, F), lambda i: (i, 0)),
        out_shape=jax.ShapeDtypeStruct((N, F), jnp.float32),
    )(parts, x, node_attr, W_dst, W_nu_r, W_last)


def kernel(x, edge_index, edge_attr, edge_embedding, node_attr, W_mlp1,
           b_mlp1, W_mlp2, b_mlp2, W_edge_proj, W_dst, W_nu, W_last):
    src = edge_index[0].astype(jnp.int32)
    dst = edge_index[1].astype(jnp.int32)
    coeff = _edge_coeff(edge_attr, edge_embedding, W_mlp1, b_mlp1, W_mlp2,
                        b_mlp2, W_edge_proj)
    zeros = jnp.zeros((NP, F), jnp.float32)
    parts = _sc_agg(x, src, dst, coeff, zeros)
    W_nu_r = jnp.transpose(W_nu, (1, 0, 2))
    return _post(parts, x, node_attr, W_dst, W_nu_r, W_last)
